# Initial kernel scaffold; baseline (speedup 1.0000x reference)
import jax
import jax.numpy as jnp
from jax.experimental import pallas as pl


def kernel(feats, lens, weights):
    raise NotImplementedError("write your pallas kernel here")


# speedup vs baseline: 5.0882x; 5.0882x over previous
import functools

import jax
import jax.numpy as jnp
from jax import lax
from jax.experimental import pallas as pl
from jax.experimental.pallas import tpu as pltpu

_K = 64
_B = 16
_START = 0
_END = 63
_NEG = -10000.0


def _fwd_body(feats_ref, lens_ref, w_ref, bptr_ref, score_ref, btag_ref, fv_ref):
    t = pl.program_id(0)
    n_t = pl.num_programs(0)

    @pl.when(t == 0)
    def _init():
        col = lax.broadcasted_iota(jnp.int32, (_B, _K), 1)
        fv_ref[...] = jnp.where(col == _START, 0.0, _NEG).astype(jnp.float32)

    fv = fv_ref[...]
    w = w_ref[...]
    s = fv[:, None, :] + w[None, :, :]
    vmax = jnp.max(s, axis=2)
    iota = lax.broadcasted_iota(jnp.int32, (_B, _K, _K), 2)
    bp = jnp.min(jnp.where(s == vmax[:, :, None], iota, _K), axis=2)
    bptr_ref[...] = bp[None].astype(jnp.int32)

    f_t = feats_ref[...][0]
    active = t < lens_ref[...]
    fv_new = jnp.where(active, vmax + f_t, fv)
    fv_ref[...] = fv_new

    @pl.when(t == n_t - 1)
    def _final():
        term = fv_new + w[_END, :][None, :]
        score_ref[...] = jnp.max(term, axis=1, keepdims=True)
        tmax = jnp.max(term, axis=1, keepdims=True)
        ic = lax.broadcasted_iota(jnp.int32, (_B, _K), 1)
        btag_ref[...] = jnp.min(jnp.where(term == tmax, ic, _K), axis=1,
                                keepdims=True)


def _bwd_body(bptr_ref, btag_ref, lens_ref, path_ref):
    n_t = path_ref.shape[0]
    lens = lens_ref[...]
    iota = lax.broadcasted_iota(jnp.int32, (_B, _K), 1)

    def step(i, tag):
        t = n_t - 1 - i
        active = t < lens
        path_ref[pl.ds(t, 1)] = jnp.where(active, tag, -1)[None]
        bp = bptr_ref[pl.ds(t, 1)][0]
        picked = jnp.max(jnp.where(iota == tag, bp, 0), axis=1, keepdims=True)
        return jnp.where(active, picked, tag)

    lax.fori_loop(0, n_t, step, btag_ref[...])


@jax.jit
def kernel(feats, lens, weights):
    B, T, K = feats.shape
    feats_t = jnp.transpose(feats, (1, 0, 2))
    lens2 = lens.reshape(B, 1)

    fwd = pl.pallas_call(
        _fwd_body,
        grid=(T,),
        in_specs=[
            pl.BlockSpec((1, B, K), lambda t: (t, 0, 0)),
            pl.BlockSpec((B, 1), lambda t: (0, 0)),
            pl.BlockSpec((K, K), lambda t: (0, 0)),
        ],
        out_specs=[
            pl.BlockSpec((1, B, K), lambda t: (t, 0, 0)),
            pl.BlockSpec((B, 1), lambda t: (0, 0)),
            pl.BlockSpec((B, 1), lambda t: (0, 0)),
        ],
        out_shape=[
            jax.ShapeDtypeStruct((T, B, K), jnp.int32),
            jax.ShapeDtypeStruct((B, 1), jnp.float32),
            jax.ShapeDtypeStruct((B, 1), jnp.int32),
        ],
        scratch_shapes=[pltpu.VMEM((B, K), jnp.float32)],
    )
    bptrs, score, btag = fwd(feats_t, lens2, weights)

    bwd = pl.pallas_call(
        _bwd_body,
        out_shape=jax.ShapeDtypeStruct((T, B, 1), jnp.int32),
    )
    paths_tb = bwd(bptrs, btag, lens2)

    return score.reshape(B), paths_tb[:, :, 0].T


# speedup vs baseline: 5.6001x; 1.1006x over previous
import functools

import jax
import jax.numpy as jnp
from jax import lax
from jax.experimental import pallas as pl
from jax.experimental.pallas import tpu as pltpu

_K = 64
_B = 16
_START = 0
_END = 63
_NEG = -10000.0


def _fwd_body(feats_ref, lens_ref, w_ref, fvh_ref, score_ref, btag_ref, fv_ref):
    t = pl.program_id(0)
    n_t = pl.num_programs(0)

    @pl.when(t == 0)
    def _init():
        col = lax.broadcasted_iota(jnp.int32, (_B, _K), 1)
        fv_ref[...] = jnp.where(col == _START, 0.0, _NEG).astype(jnp.float32)

    fv = fv_ref[...]
    fvh_ref[...] = fv[None]
    w = w_ref[...]
    s = fv[:, None, :] + w[None, :, :]
    vmax = jnp.max(s, axis=2)
    f_t = feats_ref[...][0]
    active = t < lens_ref[...]
    fv_new = jnp.where(active, vmax + f_t, fv)
    fv_ref[...] = fv_new

    @pl.when(t == n_t - 1)
    def _final():
        term = fv_new + w[_END, :][None, :]
        tmax = jnp.max(term, axis=1, keepdims=True)
        score_ref[...] = tmax
        ic = lax.broadcasted_iota(jnp.int32, (_B, _K), 1)
        btag_ref[...] = jnp.min(jnp.where(term == tmax, ic, _K), axis=1,
                                keepdims=True)


def _bwd_body(fvh_ref, wb_ref, btag_ref, lens_ref, path_ref):
    n_t = path_ref.shape[0]
    lens = lens_ref[...]
    wb = wb_ref[...]
    iota = lax.broadcasted_iota(jnp.int32, (_B, _K), 1)
    iota_f = iota.astype(jnp.float32)

    def step(i, tag):
        t = n_t - 1 - i
        active = t < lens
        path_ref[pl.ds(t, 1)] = jnp.where(active, tag, -1)[None]
        oh = (iota == tag).astype(jnp.int8)
        r = [jnp.dot(oh, wb[k], preferred_element_type=jnp.int32)
             for k in range(4)]
        bits = ((r[0] & 0xFF) | ((r[1] & 0xFF) << 8)
                | ((r[2] & 0xFF) << 16) | ((r[3] & 0xFF) << 24))
        wsel = lax.bitcast_convert_type(bits, jnp.float32)
        sc = fvh_ref[pl.ds(t, 1)][0] + wsel
        m = jnp.max(sc, axis=1, keepdims=True)
        picked = jnp.min(jnp.where(sc == m, iota_f, float(_K)), axis=1,
                         keepdims=True).astype(jnp.int32)
        return jnp.where(active, picked, tag)

    lax.fori_loop(0, n_t, step, btag_ref[...])


@jax.jit
def kernel(feats, lens, weights):
    B, T, K = feats.shape
    feats_t = jnp.transpose(feats, (1, 0, 2))
    lens2 = lens.reshape(B, 1)

    fwd = pl.pallas_call(
        _fwd_body,
        grid=(T,),
        in_specs=[
            pl.BlockSpec((1, B, K), lambda t: (t, 0, 0)),
            pl.BlockSpec((B, 1), lambda t: (0, 0)),
            pl.BlockSpec((K, K), lambda t: (0, 0)),
        ],
        out_specs=[
            pl.BlockSpec((1, B, K), lambda t: (t, 0, 0)),
            pl.BlockSpec((B, 1), lambda t: (0, 0)),
            pl.BlockSpec((B, 1), lambda t: (0, 0)),
        ],
        out_shape=[
            jax.ShapeDtypeStruct((T, B, K), jnp.float32),
            jax.ShapeDtypeStruct((B, 1), jnp.float32),
            jax.ShapeDtypeStruct((B, 1), jnp.int32),
        ],
        scratch_shapes=[pltpu.VMEM((B, K), jnp.float32)],
    )
    fvh, score, btag = fwd(feats_t, lens2, weights)

    w_bits = lax.bitcast_convert_type(weights, jnp.int32)
    w_bytes = jnp.stack([((w_bits >> (8 * k)) & 0xFF).astype(jnp.int8)
                         for k in range(4)])

    bwd = pl.pallas_call(
        _bwd_body,
        out_shape=jax.ShapeDtypeStruct((T, B, 1), jnp.int32),
    )
    paths_tb = bwd(fvh, w_bytes, btag, lens2)

    return score.reshape(B), paths_tb[:, :, 0].T


# speedup vs baseline: 7.0201x; 1.2536x over previous
import functools

import jax
import jax.numpy as jnp
from jax import lax
from jax.experimental import pallas as pl
from jax.experimental.pallas import tpu as pltpu

_K = 64
_B = 16
_START = 0
_END = 63
_NEG = -10000.0
_CF = 8
_CB = 8


def _fwd_body(feats_ref, lens_ref, w_ref, fvh_ref, score_ref, btag_ref, fv_ref):
    tau = pl.program_id(0)
    n_tau = pl.num_programs(0)

    @pl.when(tau == 0)
    def _init():
        col = lax.broadcasted_iota(jnp.int32, (_B, _K), 1)
        fv_ref[...] = jnp.where(col == _START, 0.0, _NEG).astype(jnp.float32)

    w = w_ref[...]
    lens = lens_ref[...]
    t0 = tau * _CF
    fv = fv_ref[...]
    for c in range(_CF):
        fvh_ref[pl.ds(c, 1)] = fv[None]
        s = fv[:, None, :] + w[None, :, :]
        vmax = jnp.max(s, axis=2)
        active = (t0 + c) < lens
        fv = jnp.where(active, vmax + feats_ref[pl.ds(c, 1)][0], fv)
    fv_ref[...] = fv

    @pl.when(tau == n_tau - 1)
    def _final():
        term = fv + w[_END, :][None, :]
        tmax = jnp.max(term, axis=1, keepdims=True)
        score_ref[...] = tmax
        ic = lax.broadcasted_iota(jnp.int32, (_B, _K), 1)
        btag_ref[...] = jnp.min(jnp.where(term == tmax, ic, _K), axis=1,
                                keepdims=True)


def _bwd_body(fvh_ref, wb_ref, btag_ref, lens_ref, path_ref):
    n_t = path_ref.shape[0]
    lens = lens_ref[...]
    wb = wb_ref[...]
    iota = lax.broadcasted_iota(jnp.int32, (_B, _K), 1)
    iota_f = iota.astype(jnp.float32)

    def chase(t, tag):
        active = t < lens
        path_ref[pl.ds(t, 1)] = jnp.where(active, tag, -1)[None]
        oh = (iota == tag).astype(jnp.int8)
        r = [jnp.dot(oh, wb[k], preferred_element_type=jnp.int32)
             for k in range(4)]
        bits = ((r[0] & 0xFF) | ((r[1] & 0xFF) << 8)
                | ((r[2] & 0xFF) << 16) | ((r[3] & 0xFF) << 24))
        wsel = lax.bitcast_convert_type(bits, jnp.float32)
        sc = fvh_ref[pl.ds(t, 1)][0] + wsel
        m = jnp.max(sc, axis=1, keepdims=True)
        picked = jnp.min(jnp.where(sc == m, iota_f, float(_K)), axis=1,
                         keepdims=True).astype(jnp.int32)
        return jnp.where(active, picked, tag)

    def block(i, tag):
        t_top = n_t - 1 - i * _CB
        for c in range(_CB):
            tag = chase(t_top - c, tag)
        return tag

    lax.fori_loop(0, n_t // _CB, block, btag_ref[...])


@jax.jit
def kernel(feats, lens, weights):
    B, T, K = feats.shape
    feats_t = jnp.transpose(feats, (1, 0, 2))
    lens2 = lens.reshape(B, 1)

    fwd = pl.pallas_call(
        _fwd_body,
        grid=(T // _CF,),
        in_specs=[
            pl.BlockSpec((_CF, B, K), lambda t: (t, 0, 0)),
            pl.BlockSpec((B, 1), lambda t: (0, 0)),
            pl.BlockSpec((K, K), lambda t: (0, 0)),
        ],
        out_specs=[
            pl.BlockSpec((_CF, B, K), lambda t: (t, 0, 0)),
            pl.BlockSpec((B, 1), lambda t: (0, 0)),
            pl.BlockSpec((B, 1), lambda t: (0, 0)),
        ],
        out_shape=[
            jax.ShapeDtypeStruct((T, B, K), jnp.float32),
            jax.ShapeDtypeStruct((B, 1), jnp.float32),
            jax.ShapeDtypeStruct((B, 1), jnp.int32),
        ],
        scratch_shapes=[pltpu.VMEM((B, K), jnp.float32)],
    )
    fvh, score, btag = fwd(feats_t, lens2, weights)

    w_bits = lax.bitcast_convert_type(weights, jnp.int32)
    w_bytes = jnp.stack([((w_bits >> (8 * k)) & 0xFF).astype(jnp.int8)
                         for k in range(4)])

    bwd = pl.pallas_call(
        _bwd_body,
        out_shape=jax.ShapeDtypeStruct((T, B, 1), jnp.int32),
    )
    paths_tb = bwd(fvh, w_bytes, btag, lens2)

    return score.reshape(B), paths_tb[:, :, 0].T


# speedup vs baseline: 8.0728x; 1.1499x over previous
import functools

import jax
import jax.numpy as jnp
from jax import lax
from jax.experimental import pallas as pl
from jax.experimental.pallas import tpu as pltpu

_K = 64
_B = 16
_START = 0
_END = 63
_NEG = -10000.0
_CF = 8
_CB = 8


def _fwd_body(feats_ref, lens_ref, w_ref, fvh_ref, score_ref, btag_ref, fv_ref):
    tau = pl.program_id(0)
    n_tau = pl.num_programs(0)

    @pl.when(tau == 0)
    def _init():
        col = lax.broadcasted_iota(jnp.int32, (_B, _K), 1)
        fv_ref[...] = jnp.where(col == _START, 0.0, _NEG).astype(jnp.float32)

    w = w_ref[...]
    lens = lens_ref[...]
    t0 = tau * _CF
    fv = fv_ref[...]
    for c in range(_CF):
        fvh_ref[pl.ds(c, 1)] = fv[None]
        s = fv[:, None, :] + w[None, :, :]
        vmax = jnp.max(s, axis=2)
        active = (t0 + c) < lens
        fv = jnp.where(active, vmax + feats_ref[pl.ds(c, 1)][0], fv)
    fv_ref[...] = fv

    @pl.when(tau == n_tau - 1)
    def _final():
        term = fv + w[_END, :][None, :]
        tmax = jnp.max(term, axis=1, keepdims=True)
        score_ref[...] = tmax
        ic = lax.broadcasted_iota(jnp.int32, (_B, _K), 1)
        btag_ref[...] = jnp.min(jnp.where(term == tmax, ic, _K), axis=1,
                                keepdims=True)


def _bwd_body(fvh_ref, w_ref, btag_ref, lens_ref, path_ref):
    n_t = path_ref.shape[0]
    lens = lens_ref[...]
    w = w_ref[...]
    iota_f = lax.broadcasted_iota(jnp.int32, (_B, _K), 1).astype(jnp.float32)
    rows = [jnp.broadcast_to(w[k:k + 1, :], (_B, _K)) for k in range(_K)]

    def chase(t, tag):
        active = t < lens
        path_ref[pl.ds(t, 1)] = jnp.where(active, tag, -1)[None]
        cur = rows
        for bit in range(6):
            m = (tag & (1 << bit)) != 0
            cur = [jnp.where(m, cur[2 * i + 1], cur[2 * i])
                   for i in range(len(cur) // 2)]
        wsel = cur[0]
        sc = fvh_ref[pl.ds(t, 1)][0] + wsel
        m = jnp.max(sc, axis=1, keepdims=True)
        picked = jnp.min(jnp.where(sc == m, iota_f, float(_K)), axis=1,
                         keepdims=True).astype(jnp.int32)
        return jnp.where(active, picked, tag)

    def block(i, tag):
        t_top = n_t - 1 - i * _CB
        for c in range(_CB):
            tag = chase(t_top - c, tag)
        return tag

    lax.fori_loop(0, n_t // _CB, block, btag_ref[...])


@jax.jit
def kernel(feats, lens, weights):
    B, T, K = feats.shape
    feats_t = jnp.transpose(feats, (1, 0, 2))
    lens2 = lens.reshape(B, 1)

    fwd = pl.pallas_call(
        _fwd_body,
        grid=(T // _CF,),
        in_specs=[
            pl.BlockSpec((_CF, B, K), lambda t: (t, 0, 0)),
            pl.BlockSpec((B, 1), lambda t: (0, 0)),
            pl.BlockSpec((K, K), lambda t: (0, 0)),
        ],
        out_specs=[
            pl.BlockSpec((_CF, B, K), lambda t: (t, 0, 0)),
            pl.BlockSpec((B, 1), lambda t: (0, 0)),
            pl.BlockSpec((B, 1), lambda t: (0, 0)),
        ],
        out_shape=[
            jax.ShapeDtypeStruct((T, B, K), jnp.float32),
            jax.ShapeDtypeStruct((B, 1), jnp.float32),
            jax.ShapeDtypeStruct((B, 1), jnp.int32),
        ],
        scratch_shapes=[pltpu.VMEM((B, K), jnp.float32)],
    )
    fvh, score, btag = fwd(feats_t, lens2, weights)

    bwd = pl.pallas_call(
        _bwd_body,
        out_shape=jax.ShapeDtypeStruct((T, B, 1), jnp.int32),
    )
    paths_tb = bwd(fvh, weights, btag, lens2)

    return score.reshape(B), paths_tb[:, :, 0].T


# speedup vs baseline: 8.6353x; 1.0697x over previous
import functools

import jax
import jax.numpy as jnp
from jax import lax
from jax.experimental import pallas as pl
from jax.experimental.pallas import tpu as pltpu

_K = 64
_B = 16
_START = 0
_END = 63
_NEG = -10000.0
_CF = 8
_CB = 8
_G = 2


def _fwd_body(feats_ref, lens_ref, w_ref, fvh_ref, score_ref, btag_ref, fv_ref):
    tau = pl.program_id(0)
    n_tau = pl.num_programs(0)

    @pl.when(tau == 0)
    def _init():
        col = lax.broadcasted_iota(jnp.int32, (_B, _K), 1)
        fv_ref[...] = jnp.where(col == _START, 0.0, _NEG).astype(jnp.float32)

    w = w_ref[...]
    lens = lens_ref[...]
    t0 = tau * _CF
    bg = _B // _G
    fvs = [fv_ref[pl.ds(g * bg, bg)] for g in range(_G)]
    lens_g = [lens[g * bg:(g + 1) * bg] for g in range(_G)]
    for c in range(_CF):
        for g in range(_G):
            fv = fvs[g]
            fvh_ref[pl.ds(c, 1), pl.ds(g * bg, bg)] = fv[None]
            s = fv[:, None, :] + w[None, :, :]
            vmax = jnp.max(s, axis=2)
            active = (t0 + c) < lens_g[g]
            f_t = feats_ref[pl.ds(c, 1), pl.ds(g * bg, bg)][0]
            fvs[g] = jnp.where(active, vmax + f_t, fv)
    for g in range(_G):
        fv_ref[pl.ds(g * bg, bg)] = fvs[g]

    @pl.when(tau == n_tau - 1)
    def _final():
        term = fv_ref[...] + w[_END, :][None, :]
        tmax = jnp.max(term, axis=1, keepdims=True)
        score_ref[...] = tmax
        ic = lax.broadcasted_iota(jnp.int32, (_B, _K), 1)
        btag_ref[...] = jnp.min(jnp.where(term == tmax, ic, _K), axis=1,
                                keepdims=True)


def _bwd_body(fvh_ref, w_ref, btag_ref, lens_ref, path_ref):
    n_t = path_ref.shape[0]
    lens_r = jnp.broadcast_to(lens_ref[...], (_B, _K))
    w = w_ref[...]
    iota_f = lax.broadcasted_iota(jnp.int32, (_B, _K), 1).astype(jnp.float32)
    rows = [jnp.broadcast_to(w[k:k + 1, :], (_B, _K)) for k in range(_K)]

    def chase(t, tagr):
        active = t < lens_r
        path_ref[pl.ds(t, 1)] = jnp.where(active[:, :1], tagr[:, :1], -1)[None]
        cur = rows
        for bit in range(6):
            m = (tagr & (1 << bit)) != 0
            cur = [jnp.where(m, cur[2 * i + 1], cur[2 * i])
                   for i in range(len(cur) // 2)]
        sc = fvh_ref[pl.ds(t, 1)][0] + cur[0]
        m2 = jnp.broadcast_to(jnp.max(sc, axis=1, keepdims=True), (_B, _K))
        picked = jnp.broadcast_to(
            jnp.min(jnp.where(sc == m2, iota_f, float(_K)), axis=1,
                    keepdims=True), (_B, _K)).astype(jnp.int32)
        return jnp.where(active, picked, tagr)

    def block(i, tagr):
        t_top = n_t - 1 - i * _CB
        for c in range(_CB):
            tagr = chase(t_top - c, tagr)
        return tagr

    lax.fori_loop(0, n_t // _CB, block,
                  jnp.broadcast_to(btag_ref[...], (_B, _K)))


@jax.jit
def kernel(feats, lens, weights):
    B, T, K = feats.shape
    feats_t = jnp.transpose(feats, (1, 0, 2))
    lens2 = lens.reshape(B, 1)

    fwd = pl.pallas_call(
        _fwd_body,
        grid=(T // _CF,),
        in_specs=[
            pl.BlockSpec((_CF, B, K), lambda t: (t, 0, 0)),
            pl.BlockSpec((B, 1), lambda t: (0, 0)),
            pl.BlockSpec((K, K), lambda t: (0, 0)),
        ],
        out_specs=[
            pl.BlockSpec((_CF, B, K), lambda t: (t, 0, 0)),
            pl.BlockSpec((B, 1), lambda t: (0, 0)),
            pl.BlockSpec((B, 1), lambda t: (0, 0)),
        ],
        out_shape=[
            jax.ShapeDtypeStruct((T, B, K), jnp.float32),
            jax.ShapeDtypeStruct((B, 1), jnp.float32),
            jax.ShapeDtypeStruct((B, 1), jnp.int32),
        ],
        scratch_shapes=[pltpu.VMEM((B, K), jnp.float32)],
    )
    fvh, score, btag = fwd(feats_t, lens2, weights)

    bwd = pl.pallas_call(
        _bwd_body,
        out_shape=jax.ShapeDtypeStruct((T, B, 1), jnp.int32),
    )
    paths_tb = bwd(fvh, weights, btag, lens2)

    return score.reshape(B), paths_tb[:, :, 0].T


# speedup vs baseline: 9.4227x; 1.0912x over previous
import functools

import jax
import jax.numpy as jnp
from jax import lax
from jax.experimental import pallas as pl
from jax.experimental.pallas import tpu as pltpu

_K = 64
_B = 16
_START = 0
_END = 63
_NEG = -10000.0
_CF = 8
_CB = 16
_G = 2


def _fwd_body(lens_sref, lens_ref, feats_ref, w_ref, fvh_ref, score_ref,
              btag_ref, *fv_refs):
    tau = pl.program_id(0)
    n_tau = pl.num_programs(0)
    bg = _B // _G

    @pl.when(tau == 0)
    def _init():
        col = lax.broadcasted_iota(jnp.int32, (bg, _K), 1)
        init = jnp.where(col == _START, 0.0, _NEG).astype(jnp.float32)
        for g in range(_G):
            fv_refs[g][...] = init

    w = w_ref[...]
    lens = lens_ref[...]
    t0 = tau * _CF
    for g in range(_G):
        gmax = lens_sref[g * bg]

        @pl.when(t0 < gmax)
        def _grp(g=g):
            lens_g = lens[g * bg:(g + 1) * bg]
            fv = fv_refs[g][...]
            for c in range(_CF):
                fvh_ref[pl.ds(c, 1), pl.ds(g * bg, bg)] = fv[None]
                s = fv[:, None, :] + w[None, :, :]
                vmax = jnp.max(s, axis=2)
                active = (t0 + c) < lens_g
                f_t = feats_ref[pl.ds(c, 1), pl.ds(g * bg, bg)][0]
                fv = jnp.where(active, vmax + f_t, fv)
            fv_refs[g][...] = fv

    @pl.when(tau == n_tau - 1)
    def _final():
        fv_all = jnp.concatenate([r[...] for r in fv_refs], axis=0)
        term = fv_all + w[_END, :][None, :]
        tmax = jnp.max(term, axis=1, keepdims=True)
        score_ref[...] = tmax
        ic = lax.broadcasted_iota(jnp.int32, (_B, _K), 1)
        btag_ref[...] = jnp.min(jnp.where(term == tmax, ic, _K), axis=1,
                                keepdims=True)


def _bwd_body(fvh_ref, w_ref, btag_ref, lens_ref, path_ref):
    n_t = path_ref.shape[0]
    lens_r = jnp.broadcast_to(lens_ref[...], (_B, _K))
    w = w_ref[...]
    iota_f = lax.broadcasted_iota(jnp.int32, (_B, _K), 1).astype(jnp.float32)
    rows = [jnp.broadcast_to(w[k:k + 1, :], (_B, _K)) for k in range(_K)]

    def chase(t, tagr):
        active = t < lens_r
        path_ref[pl.ds(t, 1)] = jnp.where(active[:, :1], tagr[:, :1], -1)[None]
        cur = rows
        for bit in range(6):
            m = (tagr & (1 << bit)) != 0
            cur = [jnp.where(m, cur[2 * i + 1], cur[2 * i])
                   for i in range(len(cur) // 2)]
        sc = fvh_ref[pl.ds(t, 1)][0] + cur[0]
        m2 = jnp.broadcast_to(jnp.max(sc, axis=1, keepdims=True), (_B, _K))
        picked = jnp.broadcast_to(
            jnp.min(jnp.where(sc == m2, iota_f, float(_K)), axis=1,
                    keepdims=True), (_B, _K)).astype(jnp.int32)
        return jnp.where(active, picked, tagr)

    def block(i, tagr):
        t_top = n_t - 1 - i * _CB
        for c in range(_CB):
            tagr = chase(t_top - c, tagr)
        return tagr

    lax.fori_loop(0, n_t // _CB, block,
                  jnp.broadcast_to(btag_ref[...], (_B, _K)))


@jax.jit
def kernel(feats, lens, weights):
    B, T, K = feats.shape
    perm = jnp.argsort(-lens)
    inv = jnp.argsort(perm)
    lens_p = lens[perm]
    feats_t = jnp.transpose(feats[perm], (1, 0, 2))
    lens2 = lens_p.reshape(B, 1)

    fwd = pl.pallas_call(
        _fwd_body,
        grid=(T // _CF,),
        in_specs=[
            pl.BlockSpec(memory_space=pltpu.SMEM),
            pl.BlockSpec((B, 1), lambda t: (0, 0)),
            pl.BlockSpec((_CF, B, K), lambda t: (t, 0, 0)),
            pl.BlockSpec((K, K), lambda t: (0, 0)),
        ],
        out_specs=[
            pl.BlockSpec((_CF, B, K), lambda t: (t, 0, 0)),
            pl.BlockSpec((B, 1), lambda t: (0, 0)),
            pl.BlockSpec((B, 1), lambda t: (0, 0)),
        ],
        out_shape=[
            jax.ShapeDtypeStruct((T, B, K), jnp.float32),
            jax.ShapeDtypeStruct((B, 1), jnp.float32),
            jax.ShapeDtypeStruct((B, 1), jnp.int32),
        ],
        scratch_shapes=[pltpu.VMEM((B // _G, K), jnp.float32)
                        for _ in range(_G)],
    )
    fvh, score, btag = fwd(lens_p, lens2, feats_t, weights)

    bwd = pl.pallas_call(
        _bwd_body,
        out_shape=jax.ShapeDtypeStruct((T, B, 1), jnp.int32),
    )
    paths_tb = bwd(fvh, weights, btag, lens2)

    return score.reshape(B)[inv], paths_tb[:, :, 0].T[inv]


# speedup vs baseline: 9.4236x; 1.0001x over previous
import jax
import jax.numpy as jnp
from jax import lax
from jax.experimental import pallas as pl
from jax.experimental.pallas import tpu as pltpu

_K = 64
_B = 16
_START = 0
_END = 63
_NEG = -10000.0
_CF = 8
_CB = 16
_G = 2


def _fwd_body(lens_sref, lens_ref, feats_ref, w_ref, fvh_ref, score_ref,
              btag_ref, *fv_refs):
    tau = pl.program_id(0)
    n_tau = pl.num_programs(0)
    bg = _B // _G

    @pl.when(tau == 0)
    def _init():
        col = lax.broadcasted_iota(jnp.int32, (bg, _K), 1)
        init = jnp.where(col == _START, 0.0, _NEG).astype(jnp.float32)
        for g in range(_G):
            fv_refs[g][...] = init

    w = w_ref[...]
    lens = lens_ref[...]
    t0 = tau * _CF
    for g in range(_G):
        gmax = lens_sref[g * bg]

        @pl.when(t0 < gmax)
        def _grp(g=g):
            lens_g = lens[g * bg:(g + 1) * bg]
            fv = fv_refs[g][...]
            for c in range(_CF):
                fvh_ref[pl.ds(c, 1), pl.ds(g * bg, bg)] = fv[None]
                s = fv[:, None, :] + w[None, :, :]
                vmax = jnp.max(s, axis=2)
                active = (t0 + c) < lens_g
                f_t = feats_ref[pl.ds(c, 1), pl.ds(g * bg, bg)][0]
                fv = jnp.where(active, vmax + f_t, fv)
            fv_refs[g][...] = fv

    @pl.when(tau == n_tau - 1)
    def _final():
        fv_all = jnp.concatenate([r[...] for r in fv_refs], axis=0)
        term = fv_all + w[_END, :][None, :]
        tmax = jnp.max(term, axis=1, keepdims=True)
        score_ref[...] = tmax
        ic = lax.broadcasted_iota(jnp.int32, (_B, _K), 1)
        btag_ref[...] = jnp.min(jnp.where(term == tmax, ic, _K), axis=1,
                                keepdims=True)


def _bwd_body(fvh_ref, w_ref, btag_ref, lens_ref, path_ref):
    n_t = path_ref.shape[0]
    lens_r = jnp.broadcast_to(lens_ref[...], (_B, _K))
    w = w_ref[...]
    iota_f = lax.broadcasted_iota(jnp.int32, (_B, _K), 1).astype(jnp.float32)
    rows = [jnp.broadcast_to(w[k:k + 1, :], (_B, _K)) for k in range(_K)]

    def chase(t, tagr):
        active = t < lens_r
        path_ref[pl.ds(t, 1)] = jnp.where(active[:, :1], tagr[:, :1], -1)[None]
        cur = rows
        for bit in range(6):
            m = (tagr & (1 << bit)) != 0
            cur = [jnp.where(m, cur[2 * i + 1], cur[2 * i])
                   for i in range(len(cur) // 2)]
        sc = fvh_ref[pl.ds(t, 1)][0] + cur[0]
        m2 = jnp.broadcast_to(jnp.max(sc, axis=1, keepdims=True), (_B, _K))
        picked = jnp.broadcast_to(
            jnp.min(jnp.where(sc == m2, iota_f, float(_K)), axis=1,
                    keepdims=True), (_B, _K)).astype(jnp.int32)
        return jnp.where(active, picked, tagr)

    def block(i, tagr):
        t_top = n_t - 1 - i * _CB
        for c in range(_CB):
            tagr = chase(t_top - c, tagr)
        return tagr

    lax.fori_loop(0, n_t // _CB, block,
                  jnp.broadcast_to(btag_ref[...], (_B, _K)))


@jax.jit
def kernel(feats, lens, weights):
    B, T, K = feats.shape
    perm = jnp.argsort(-lens)
    inv = jnp.argsort(perm)
    lens_p = lens[perm]
    feats_t = jnp.transpose(feats[perm], (1, 0, 2))
    lens2 = lens_p.reshape(B, 1)

    fwd = pl.pallas_call(
        _fwd_body,
        grid=(T // _CF,),
        in_specs=[
            pl.BlockSpec(memory_space=pltpu.SMEM),
            pl.BlockSpec((B, 1), lambda t: (0, 0)),
            pl.BlockSpec((_CF, B, K), lambda t: (t, 0, 0)),
            pl.BlockSpec((K, K), lambda t: (0, 0)),
        ],
        out_specs=[
            pl.BlockSpec((_CF, B, K), lambda t: (t, 0, 0)),
            pl.BlockSpec((B, 1), lambda t: (0, 0)),
            pl.BlockSpec((B, 1), lambda t: (0, 0)),
        ],
        out_shape=[
            jax.ShapeDtypeStruct((T, B, K), jnp.float32),
            jax.ShapeDtypeStruct((B, 1), jnp.float32),
            jax.ShapeDtypeStruct((B, 1), jnp.int32),
        ],
        scratch_shapes=[pltpu.VMEM((B // _G, K), jnp.float32)
                        for _ in range(_G)],
    )
    fvh, score, btag = fwd(lens_p, lens2, feats_t, weights)

    bwd = pl.pallas_call(
        _bwd_body,
        out_shape=jax.ShapeDtypeStruct((T, B, 1), jnp.int32),
    )
    paths_tb = bwd(fvh, weights, btag, lens2)

    return score.reshape(B)[inv], paths_tb[:, :, 0].T[inv]
